# baseline (speedup 1.0000x reference)
import functools

import jax
import jax.numpy as jnp
from jax import lax
from jax.experimental import pallas as pl
from jax.experimental.pallas import tpu as pltpu
from jax.experimental.pallas import tpu_sc as plsc

N = 64
K = 64
B = 32
P = 128
NN = N * N
CBLK = 256
J = NN // CBLK
_L = 16


def _wg_ef_body(w_ref, g_ref, wg_ref, ef_ref):
    w = w_ref[...]
    w2 = (w * w).astype(jnp.bfloat16)
    wg = jnp.dot(w2, g_ref[...].astype(jnp.bfloat16),
                 preferred_element_type=jnp.float32)
    wg_ref[...] = wg
    ef_ref[...] = jnp.exp(-wg)


def _seg_sc_body(x_hbm, mask_hbm, xi_hbm, s_hbm, x_v, acc_v, mask_s, xi_s):
    b = lax.axis_index("s") * 2 + lax.axis_index("c")
    pltpu.sync_copy(x_hbm.at[b], x_v)
    pltpu.sync_copy(mask_hbm.at[b], mask_s)
    pltpu.sync_copy(xi_hbm.at[b], xi_s)

    def zero_body(i, _):
        acc_v[pl.ds(i * _L, _L)] = jnp.zeros((_L,), jnp.float32)
        return 0
    lax.fori_loop(0, K * N // _L, zero_body, 0)

    def chunk_body(ch, _):
        mvec = mask_s[pl.ds(ch * _L, _L)]
        xvec = xi_s[pl.ds(ch * _L, _L)] * jnp.float32(0.1)
        for j in range(_L):
            base = mvec[j] * N
            xi_b = xvec[j]
            tok = (ch * _L + j) * N
            for c in range(N // _L):
                col = x_v[pl.ds(tok + c * _L, _L)]
                off = base + c * _L
                acc_v[pl.ds(off, _L)] = acc_v[pl.ds(off, _L)] + (col - xi_b)
        return 0
    lax.fori_loop(0, P // _L, chunk_body, 0)

    pltpu.sync_copy(acc_v, s_hbm.at[b])


_seg_sc = functools.partial(
    pl.kernel,
    out_type=jax.ShapeDtypeStruct((B, K * N), jnp.float32),
    mesh=plsc.VectorSubcoreMesh(core_axis_name="c", subcore_axis_name="s"),
    scratch_types=[
        pltpu.VMEM((N * P,), jnp.float32),
        pltpu.VMEM((K * N,), jnp.float32),
        pltpu.VMEM((P,), jnp.int32),
        pltpu.VMEM((P,), jnp.float32),
    ],
)(_seg_sc_body)


def _z_body(s_ref, a_ref, y_ref, z_ref):
    z_ref[...] = (jnp.dot(s_ref[...], a_ref[...],
                          preferred_element_type=jnp.float32)
                  + 0.1 * y_ref[...])


def kernel(x, x_mask, x_i, y_i, weights, g):
    y2 = y_i[:, :, 0]

    wg, ef = pl.pallas_call(
        _wg_ef_body,
        grid=(J,),
        in_specs=[pl.BlockSpec((K, NN), lambda j: (0, 0)),
                  pl.BlockSpec((NN, CBLK), lambda j: (0, j))],
        out_specs=[pl.BlockSpec((K, CBLK), lambda j: (0, j)),
                   pl.BlockSpec((K, CBLK), lambda j: (0, j))],
        out_shape=[jax.ShapeDtypeStruct((K, NN), jnp.float32),
                   jax.ShapeDtypeStruct((K, NN), jnp.float32)],
    )(weights, g)

    xt = x.transpose(0, 2, 1).reshape(B, P * N)
    sflat = _seg_sc(xt, x_mask.astype(jnp.int32), x_i)

    a = ef.reshape(K, N, N).transpose(0, 2, 1).reshape(K * N, N)

    z = pl.pallas_call(
        _z_body,
        out_shape=jax.ShapeDtypeStruct((B, N), jnp.float32),
    )(sflat, a, y2)
    return (z, wg)


# speedup vs baseline: 1.0411x; 1.0411x over previous
import functools

import jax
import jax.numpy as jnp
from jax import lax
from jax.experimental import pallas as pl
from jax.experimental.pallas import tpu as pltpu
from jax.experimental.pallas import tpu_sc as plsc

N = 64
K = 64
B = 32
P = 128
NN = N * N
CBLK = 512
J = NN // CBLK
_L = 16


def _wg_ef_body(w_ref, g_ref, wg_ref, ef_ref):
    w = w_ref[...]
    w2 = (w * w).astype(jnp.bfloat16)
    wg = jnp.dot(w2, g_ref[...].astype(jnp.bfloat16),
                 preferred_element_type=jnp.float32)
    wg_ref[...] = wg
    ef_ref[...] = jnp.exp(-wg)


def _seg_sc_body(x_hbm, mask_hbm, xi_hbm, s_hbm, x_v, acc_v, mask_s, xi_s):
    b = lax.axis_index("s") * 2 + lax.axis_index("c")
    pltpu.sync_copy(x_hbm.at[b], x_v)
    pltpu.sync_copy(mask_hbm.at[b], mask_s)
    pltpu.sync_copy(xi_hbm.at[b], xi_s)

    def zero_body(i, _):
        acc_v[pl.ds(i * _L, _L)] = jnp.zeros((_L,), jnp.float32)
        return 0
    lax.fori_loop(0, K * N // _L, zero_body, 0)

    def chunk_body(ch, _):
        mvec = mask_s[pl.ds(ch * _L, _L)]
        xvec = xi_s[pl.ds(ch * _L, _L)] * jnp.float32(0.1)
        for j in range(_L):
            base = mvec[j] * N
            xi_b = xvec[j]
            tok = (ch * _L + j) * N
            for c in range(N // _L):
                col = x_v[pl.ds(tok + c * _L, _L)]
                off = base + c * _L
                acc_v[pl.ds(off, _L)] = acc_v[pl.ds(off, _L)] + (col - xi_b)
        return 0
    lax.fori_loop(0, P // _L, chunk_body, 0)

    pltpu.sync_copy(acc_v, s_hbm.at[b])


_seg_sc = functools.partial(
    pl.kernel,
    out_type=jax.ShapeDtypeStruct((B, K * N), jnp.float32),
    mesh=plsc.VectorSubcoreMesh(core_axis_name="c", subcore_axis_name="s"),
    scratch_types=[
        pltpu.VMEM((N * P,), jnp.float32),
        pltpu.VMEM((K * N,), jnp.float32),
        pltpu.VMEM((P,), jnp.int32),
        pltpu.VMEM((P,), jnp.float32),
    ],
)(_seg_sc_body)


def _z_body(s_ref, a_ref, y_ref, z_ref):
    z_ref[...] = (jnp.dot(s_ref[...], a_ref[...],
                          preferred_element_type=jnp.float32)
                  + 0.1 * y_ref[...])


def kernel(x, x_mask, x_i, y_i, weights, g):
    y2 = y_i[:, :, 0]

    wg, ef = pl.pallas_call(
        _wg_ef_body,
        grid=(J,),
        in_specs=[pl.BlockSpec((K, NN), lambda j: (0, 0)),
                  pl.BlockSpec((NN, CBLK), lambda j: (0, j))],
        out_specs=[pl.BlockSpec((K, CBLK), lambda j: (0, j)),
                   pl.BlockSpec((K, CBLK), lambda j: (0, j))],
        out_shape=[jax.ShapeDtypeStruct((K, NN), jnp.float32),
                   jax.ShapeDtypeStruct((K, NN), jnp.float32)],
    )(weights, g)

    xt = x.transpose(0, 2, 1).reshape(B, P * N)
    sflat = _seg_sc(xt, x_mask.astype(jnp.int32), x_i)

    a = ef.reshape(K, N, N).transpose(0, 2, 1).reshape(K * N, N)

    z = pl.pallas_call(
        _z_body,
        out_shape=jax.ShapeDtypeStruct((B, N), jnp.float32),
    )(sflat, a, y2)
    return (z, wg)


# speedup vs baseline: 1.1404x; 1.0953x over previous
import functools

import jax
import jax.numpy as jnp
from jax import lax
from jax.experimental import pallas as pl
from jax.experimental.pallas import tpu as pltpu
from jax.experimental.pallas import tpu_sc as plsc

N = 64
K = 64
B = 32
P = 128
NN = N * N
CBLK = 512
J = NN // CBLK
_L = 16


def _wg_ef_body(w_ref, g_ref, wg_ref, a_ref):
    w = w_ref[...]
    w2 = (w * w).astype(jnp.bfloat16)
    wg = jnp.dot(w2, g_ref[...].astype(jnp.bfloat16),
                 preferred_element_type=jnp.float32)
    wg_ref[...] = wg
    ef3 = jnp.exp(-wg).reshape(K, CBLK // N, N)
    a_ref[...] = jnp.transpose(ef3, (1, 0, 2)).reshape(CBLK // N, K * N)


def _seg_sc_body(x_hbm, mask_hbm, xi_hbm, s_hbm, x_v, acc_v, mask_s, xi_s):
    b = lax.axis_index("s") * 2 + lax.axis_index("c")
    pltpu.sync_copy(x_hbm.at[b], x_v)
    pltpu.sync_copy(mask_hbm.at[b], mask_s)
    pltpu.sync_copy(xi_hbm.at[b], xi_s)

    def zero_body(i, _):
        acc_v[pl.ds(i * _L, _L)] = jnp.zeros((_L,), jnp.float32)
        return 0
    lax.fori_loop(0, K * N // _L, zero_body, 0)

    def chunk_body(ch, _):
        mvec = mask_s[pl.ds(ch * _L, _L)]
        xvec = xi_s[pl.ds(ch * _L, _L)] * jnp.float32(0.1)
        for j in range(_L):
            base = mvec[j] * N
            xi_b = xvec[j]
            tok = (ch * _L + j) * N
            for c in range(N // _L):
                col = x_v[pl.ds(tok + c * _L, _L)]
                off = base + c * _L
                acc_v[pl.ds(off, _L)] = acc_v[pl.ds(off, _L)] + (col - xi_b)
        return 0
    lax.fori_loop(0, P // _L, chunk_body, 0)

    pltpu.sync_copy(acc_v, s_hbm.at[b])


_seg_sc = functools.partial(
    pl.kernel,
    out_type=jax.ShapeDtypeStruct((B, K * N), jnp.float32),
    mesh=plsc.VectorSubcoreMesh(core_axis_name="c", subcore_axis_name="s"),
    scratch_types=[
        pltpu.VMEM((N * P,), jnp.float32),
        pltpu.VMEM((K * N,), jnp.float32),
        pltpu.VMEM((P,), jnp.int32),
        pltpu.VMEM((P,), jnp.float32),
    ],
)(_seg_sc_body)


def _z_body(s_ref, a_ref, y_ref, z_ref):
    z_ref[...] = (lax.dot_general(s_ref[...], a_ref[...],
                                  (((1,), (1,)), ((), ())),
                                  preferred_element_type=jnp.float32)
                  + 0.1 * y_ref[...])


def kernel(x, x_mask, x_i, y_i, weights, g):
    y2 = y_i[:, :, 0]

    wg, a = pl.pallas_call(
        _wg_ef_body,
        grid=(J,),
        in_specs=[pl.BlockSpec((K, NN), lambda j: (0, 0)),
                  pl.BlockSpec((NN, CBLK), lambda j: (0, j))],
        out_specs=[pl.BlockSpec((K, CBLK), lambda j: (0, j)),
                   pl.BlockSpec((CBLK // N, K * N), lambda j: (j, 0))],
        out_shape=[jax.ShapeDtypeStruct((K, NN), jnp.float32),
                   jax.ShapeDtypeStruct((N, K * N), jnp.float32)],
    )(weights, g)

    xt = x.transpose(0, 2, 1).reshape(B, P * N)
    sflat = _seg_sc(xt, x_mask.astype(jnp.int32), x_i)

    z = pl.pallas_call(
        _z_body,
        out_shape=jax.ShapeDtypeStruct((B, N), jnp.float32),
    )(sflat, a, y2)
    return (z, wg)
